# baseline (device time: 75417 ns/iter reference)
import jax
import jax.numpy as jnp
from jax import lax
from jax.experimental import pallas as pl
from jax.experimental.pallas import tpu as pltpu

N_DEV = 4
B, SQ, SKV = 2, 512, 512
HQ_LOC, DH = 8, 64
DM = 768
DQ_LOC = HQ_LOC * DH
ROWS = B * SQ
CHUNK = ROWS // N_DEV


def kernel(x, Wq, K_ext, V_ext, Wo):
    i = lax.axis_index("i")
    Wq_loc = (lax.dynamic_slice(Wq, (0, i * DQ_LOC), (DM, DQ_LOC)) * 0.125
              ).astype(jnp.bfloat16)
    Wo_loc = lax.dynamic_slice(Wo, (i * DQ_LOC, 0), (DQ_LOC, DM)
                               ).astype(jnp.bfloat16)
    x16 = x.astype(jnp.bfloat16)
    K16 = K_ext.astype(jnp.bfloat16)
    V16 = V_ext.astype(jnp.bfloat16)

    def body(x_ref, wq_ref, k_ref, v_ref, wo_ref, out_ref,
             acc_ref, ctx_ref, snd_rs, rs_buf, ag_ref,
             send_sems, recv_sems):
        my = lax.axis_index("i")

        barrier_sem = pltpu.get_barrier_semaphore()
        for d in range(1, N_DEV):
            pl.semaphore_signal(
                barrier_sem, inc=1,
                device_id=(lax.rem(my + d, N_DEV),),
                device_id_type=pl.DeviceIdType.MESH,
            )
        pl.semaphore_wait(barrier_sem, N_DEV - 1)

        qi = lax.broadcasted_iota(jnp.int32, (SQ, SKV), 0)
        ki = lax.broadcasted_iota(jnp.int32, (SQ, SKV), 1)
        dd = qi - ki
        mask = ((dd <= 128) & (dd >= -128)) | (ki < 32) | (qi < 32)
        bias = jnp.where(mask, 0.0, -1e9).astype(jnp.float32)

        qs = [
            jnp.dot(x_ref[b, :, :], wq_ref[:, :],
                    preferred_element_type=jnp.float32).astype(jnp.bfloat16)
            for b in range(B)
        ]

        def rs_send(d):
            return pltpu.make_async_remote_copy(
                src_ref=snd_rs.at[d - 1],
                dst_ref=rs_buf.at[d - 1],
                send_sem=send_sems.at[d - 1],
                recv_sem=recv_sems.at[d - 1],
                device_id=(lax.rem(my + d, N_DEV),),
                device_id_type=pl.DeviceIdType.MESH,
            )

        for c in range(N_DEV):
            b = c // 2
            rq0 = (c % 2) * CHUNK
            r0 = c * CHUNK
            biasc = bias[rq0:rq0 + CHUNK, :]
            for h in range(HQ_LOC):
                qh = qs[b][rq0:rq0 + CHUNK, h * DH:(h + 1) * DH]
                kh = k_ref[b, :, h, :]
                vh = v_ref[b, :, h, :]
                s = lax.dot_general(
                    qh, kh, (((1,), (1,)), ((), ())),
                    preferred_element_type=jnp.float32)
                w = jnp.exp(s + biasc)
                denom = jnp.sum(w, axis=-1, keepdims=True)
                ctx = jnp.dot(w.astype(jnp.bfloat16), vh,
                              preferred_element_type=jnp.float32)
                ctx_ref[:, h * DH:(h + 1) * DH] = (
                    ctx / denom).astype(jnp.bfloat16)
            acc_ref[r0:r0 + CHUNK, :] = jnp.dot(
                ctx_ref[:, :], wo_ref[:, :],
                preferred_element_type=jnp.float32)

            for d in range(1, N_DEV):
                sender = (c - d) % N_DEV

                @pl.when(my == sender)
                def _(d=d):
                    snd_rs[d - 1, :, :] = acc_ref[r0:r0 + CHUNK,
                                                  :].astype(jnp.bfloat16)
                    rs_send(d).start()

            @pl.when(my == c)
            def _():
                for d in range(1, N_DEV):
                    rs_send(d).wait_recv()
                red = acc_ref[r0:r0 + CHUNK, :]
                for d in range(1, N_DEV):
                    red = red + rs_buf[d - 1, :, :].astype(jnp.float32)
                ag_ref[r0:r0 + CHUNK, :] = red.astype(jnp.bfloat16)
                for d in range(1, N_DEV):
                    pltpu.make_async_remote_copy(
                        src_ref=ag_ref.at[pl.ds(r0, CHUNK), :],
                        dst_ref=ag_ref.at[pl.ds(r0, CHUNK), :],
                        send_sem=send_sems.at[N_DEV - 1 + d - 1],
                        recv_sem=recv_sems.at[N_DEV - 1 + d - 1],
                        device_id=(lax.rem(my + d, N_DEV),),
                        device_id_type=pl.DeviceIdType.MESH,
                    ).start()

        for d in range(1, N_DEV):
            pltpu.make_async_remote_copy(
                src_ref=rs_buf.at[d - 1],
                dst_ref=rs_buf.at[d - 1],
                send_sem=send_sems.at[N_DEV - 1 + d - 1],
                recv_sem=recv_sems.at[N_DEV - 1 + d - 1],
                device_id=(my,),
                device_id_type=pl.DeviceIdType.MESH,
            ).wait_recv()

        for d in range(1, N_DEV):
            rs_send(d).wait_send()
        for d in range(1, N_DEV):
            pltpu.make_async_remote_copy(
                src_ref=rs_buf.at[d - 1],
                dst_ref=rs_buf.at[d - 1],
                send_sem=send_sems.at[N_DEV - 1 + d - 1],
                recv_sem=recv_sems.at[N_DEV - 1 + d - 1],
                device_id=(my,),
                device_id_type=pl.DeviceIdType.MESH,
            ).wait_send()

        out_ref[0, :, :] = ag_ref[pl.ds(0, SQ), :].astype(jnp.float32)
        out_ref[1, :, :] = ag_ref[pl.ds(SQ, SQ), :].astype(jnp.float32)

    return pl.pallas_call(
        body,
        out_shape=jax.ShapeDtypeStruct((B, SQ, DM), jnp.float32),
        in_specs=[pl.BlockSpec(memory_space=pltpu.VMEM)] * 5,
        out_specs=pl.BlockSpec(memory_space=pltpu.VMEM),
        scratch_shapes=[
            pltpu.VMEM((ROWS, DM), jnp.float32),
            pltpu.VMEM((CHUNK, DQ_LOC), jnp.bfloat16),
            pltpu.VMEM((N_DEV - 1, CHUNK, DM), jnp.bfloat16),
            pltpu.VMEM((N_DEV - 1, CHUNK, DM), jnp.bfloat16),
            pltpu.VMEM((ROWS, DM), jnp.bfloat16),
            pltpu.SemaphoreType.DMA((2 * (N_DEV - 1),)),
            pltpu.SemaphoreType.DMA((2 * (N_DEV - 1),)),
        ],
        compiler_params=pltpu.CompilerParams(collective_id=0),
    )(x16, Wq_loc, K16, V16, Wo_loc)


# device time: 47441 ns/iter; 1.5897x vs baseline; 1.5897x over previous
import jax
import jax.numpy as jnp
from jax import lax
from jax.experimental import pallas as pl
from jax.experimental.pallas import tpu as pltpu

N_DEV = 4
B, SQ, SKV = 2, 512, 512
HQ_LOC, DH = 8, 64
DM = 768
DQ_LOC = HQ_LOC * DH
ROWS = B * SQ
CHUNK = ROWS // N_DEV


def kernel(x, Wq, K_ext, V_ext, Wo):
    i = lax.axis_index("i")
    Wq_loc = (lax.dynamic_slice(Wq, (0, i * DQ_LOC), (DM, DQ_LOC)) * 0.125
              ).astype(jnp.bfloat16)
    Wo_loc = lax.dynamic_slice(Wo, (i * DQ_LOC, 0), (DQ_LOC, DM)
                               ).astype(jnp.bfloat16)
    x16 = x.astype(jnp.bfloat16)
    K16 = K_ext.astype(jnp.bfloat16)

    def body(x_ref, wq_ref, k_ref, v_ref, wo_ref, out_ref,
             acc_ref, ctx_ref, vaug_ref, snd_rs, rs_buf, ag_ref,
             send_sems, recv_sems):
        my = lax.axis_index("i")

        barrier_sem = pltpu.get_barrier_semaphore()
        for d in range(1, N_DEV):
            pl.semaphore_signal(
                barrier_sem, inc=1,
                device_id=(lax.rem(my + d, N_DEV),),
                device_id_type=pl.DeviceIdType.MESH,
            )
        pl.semaphore_wait(barrier_sem, N_DEV - 1)

        qi = lax.broadcasted_iota(jnp.int32, (SQ, SKV), 0)
        ki = lax.broadcasted_iota(jnp.int32, (SQ, SKV), 1)
        dd = qi - ki
        mask = ((dd <= 128) & (dd >= -128)) | (ki < 32) | (qi < 32)
        bias = jnp.where(mask, 0.0, -1e9).astype(jnp.float32)

        onescol = (lax.broadcasted_iota(jnp.int32, (SKV, DH), 1) == 0
                   ).astype(jnp.bfloat16)
        for b in range(B):
            for h in range(HQ_LOC):
                vaug_ref[b * HQ_LOC + h, :, :DH] = (
                    v_ref[b, :, h, :].astype(jnp.bfloat16))
                vaug_ref[b * HQ_LOC + h, :, DH:] = onescol

        def rs_send(d):
            c = lax.rem(my + d, N_DEV)
            return pltpu.make_async_remote_copy(
                src_ref=snd_rs.at[d - 1],
                dst_ref=rs_buf.at[d - 1],
                send_sem=send_sems.at[d - 1],
                recv_sem=recv_sems.at[d - 1],
                device_id=(c,),
                device_id_type=pl.DeviceIdType.MESH,
            )

        def rs_stage_and_send(d):
            c = lax.rem(my + d, N_DEV)
            snd_rs[d - 1, :, :] = acc_ref[pl.ds(c * CHUNK, CHUNK),
                                          :].astype(jnp.bfloat16)
            rs_send(d).start()

        for b in range(B):
            xb = x_ref[b, :, :]
            q = jnp.dot(xb, wq_ref[:, :],
                        preferred_element_type=jnp.float32
                        ).astype(jnp.bfloat16)
            for h in range(HQ_LOC):
                qh = q[:, h * DH:(h + 1) * DH]
                kh = k_ref[b, :, h, :]
                s = lax.dot_general(
                    qh, kh, (((1,), (1,)), ((), ())),
                    preferred_element_type=jnp.float32)
                w = jnp.exp(s + bias).astype(jnp.bfloat16)
                cd = jnp.dot(w, vaug_ref[b * HQ_LOC + h, :, :],
                             preferred_element_type=jnp.float32)
                ctx_ref[:, h * DH:(h + 1) * DH] = (
                    cd[:, :DH] / cd[:, DH:DH + 1]).astype(jnp.bfloat16)
            acc_ref[pl.ds(b * SQ, SQ), :] = jnp.dot(
                ctx_ref[:, :], wo_ref[:, :],
                preferred_element_type=jnp.float32)

            for d in range(1, N_DEV):
                c = lax.rem(my + d, N_DEV)
                if b == 0:
                    @pl.when(c < 2)
                    def _(d=d):
                        rs_stage_and_send(d)
                else:
                    @pl.when(c >= 2)
                    def _(d=d):
                        rs_stage_and_send(d)

        for d in range(1, N_DEV):
            rs_send(d).wait_recv()
        red = acc_ref[pl.ds(my * CHUNK, CHUNK), :]
        for d in range(1, N_DEV):
            red = red + rs_buf[d - 1, :, :].astype(jnp.float32)
        ag_ref[pl.ds(my * CHUNK, CHUNK), :] = red.astype(jnp.bfloat16)

        ag_rdmas = []
        for d in range(1, N_DEV):
            r = pltpu.make_async_remote_copy(
                src_ref=ag_ref.at[pl.ds(my * CHUNK, CHUNK), :],
                dst_ref=ag_ref.at[pl.ds(my * CHUNK, CHUNK), :],
                send_sem=send_sems.at[N_DEV - 1 + d - 1],
                recv_sem=recv_sems.at[N_DEV - 1 + d - 1],
                device_id=(lax.rem(my + d, N_DEV),),
                device_id_type=pl.DeviceIdType.MESH,
            )
            r.start()
            ag_rdmas.append(r)

        for d in range(1, N_DEV):
            src = lax.rem(my - d + N_DEV, N_DEV)
            pltpu.make_async_remote_copy(
                src_ref=ag_ref.at[pl.ds(src * CHUNK, CHUNK), :],
                dst_ref=ag_ref.at[pl.ds(src * CHUNK, CHUNK), :],
                send_sem=send_sems.at[N_DEV - 1 + d - 1],
                recv_sem=recv_sems.at[N_DEV - 1 + d - 1],
                device_id=(src,),
                device_id_type=pl.DeviceIdType.MESH,
            ).wait_recv()

        for d in range(1, N_DEV):
            rs_send(d).wait_send()
        for r in ag_rdmas:
            r.wait_send()

        out_ref[0, :, :] = ag_ref[pl.ds(0, SQ), :].astype(jnp.float32)
        out_ref[1, :, :] = ag_ref[pl.ds(SQ, SQ), :].astype(jnp.float32)

    return pl.pallas_call(
        body,
        out_shape=jax.ShapeDtypeStruct((B, SQ, DM), jnp.float32),
        in_specs=[pl.BlockSpec(memory_space=pltpu.VMEM)] * 5,
        out_specs=pl.BlockSpec(memory_space=pltpu.VMEM),
        scratch_shapes=[
            pltpu.VMEM((ROWS, DM), jnp.float32),
            pltpu.VMEM((SQ, DQ_LOC), jnp.bfloat16),
            pltpu.VMEM((B * HQ_LOC, SKV, 2 * DH), jnp.bfloat16),
            pltpu.VMEM((N_DEV - 1, CHUNK, DM), jnp.bfloat16),
            pltpu.VMEM((N_DEV - 1, CHUNK, DM), jnp.bfloat16),
            pltpu.VMEM((ROWS, DM), jnp.bfloat16),
            pltpu.SemaphoreType.DMA((2 * (N_DEV - 1),)),
            pltpu.SemaphoreType.DMA((2 * (N_DEV - 1),)),
        ],
        compiler_params=pltpu.CompilerParams(collective_id=0),
    )(x16, Wq_loc, K16, V_ext, Wo_loc)


# device time: 43787 ns/iter; 1.7224x vs baseline; 1.0834x over previous
import jax
import jax.numpy as jnp
from jax import lax
from jax.experimental import pallas as pl
from jax.experimental.pallas import tpu as pltpu

N_DEV = 4
B, SQ, SKV = 2, 512, 512
HQ_LOC, DH = 8, 64
DHP = 128
DM = 768
DQ_LOC = HQ_LOC * DH
DQP = HQ_LOC * DHP
ROWS = B * SQ
CHUNK = ROWS // N_DEV


def kernel(x, Wq, K_ext, V_ext, Wo):
    i = lax.axis_index("i")
    f16 = jnp.bfloat16

    Wq_loc = lax.dynamic_slice(Wq, (0, i * DQ_LOC), (DM, DQ_LOC)) * 0.125
    Wq_pad = jnp.pad(Wq_loc.reshape(DM, HQ_LOC, DH).astype(f16),
                     ((0, 0), (0, 0), (0, DHP - DH))).reshape(DM, DQP)
    Wo_loc = lax.dynamic_slice(Wo, (i * DQ_LOC, 0), (DQ_LOC, DM))
    Wo_pad = jnp.pad(Wo_loc.reshape(HQ_LOC, DH, DM).astype(f16),
                     ((0, 0), (0, DHP - DH), (0, 0))).reshape(DQP, DM)
    x16 = x.astype(f16)
    Kt_pad = jnp.pad(K_ext.transpose(0, 2, 3, 1).astype(f16),
                     ((0, 0), (0, 0), (0, DHP - DH), (0, 0)))
    Vt = V_ext.transpose(0, 2, 1, 3).astype(f16)
    Vaug = jnp.concatenate(
        [Vt, jnp.ones((B, HQ_LOC, SKV, 1), f16),
         jnp.zeros((B, HQ_LOC, SKV, DHP - DH - 1), f16)], axis=-1)

    def body(x_ref, wq_ref, kt_ref, va_ref, wo_ref, out_ref,
             acc_ref, ctx_ref, snd_rs, rs_buf, ag_ref,
             send_sems, recv_sems):
        my = lax.axis_index("i")

        barrier_sem = pltpu.get_barrier_semaphore()
        for d in range(1, N_DEV):
            pl.semaphore_signal(
                barrier_sem, inc=1,
                device_id=(lax.rem(my + d, N_DEV),),
                device_id_type=pl.DeviceIdType.MESH,
            )
        pl.semaphore_wait(barrier_sem, N_DEV - 1)

        qi = lax.broadcasted_iota(jnp.int32, (SQ, SKV), 0)
        ki = lax.broadcasted_iota(jnp.int32, (SQ, SKV), 1)
        dd = qi - ki
        mask = ((dd <= 128) & (dd >= -128)) | (ki < 32) | (qi < 32)
        bias = jnp.where(mask, 0.0, -1e9).astype(jnp.float32)

        def rs_send(d):
            return pltpu.make_async_remote_copy(
                src_ref=snd_rs.at[d - 1],
                dst_ref=rs_buf.at[d - 1],
                send_sem=send_sems.at[d - 1],
                recv_sem=recv_sems.at[d - 1],
                device_id=(lax.rem(my + d, N_DEV),),
                device_id_type=pl.DeviceIdType.MESH,
            )

        def rs_stage_and_send(d):
            c = lax.rem(my + d, N_DEV)
            snd_rs[d - 1, :, :] = acc_ref[pl.ds(c * CHUNK, CHUNK),
                                          :].astype(f16)
            rs_send(d).start()

        for b in range(B):
            q = jnp.dot(x_ref[b, :, :], wq_ref[:, :],
                        preferred_element_type=jnp.float32
                        ).astype(f16)
            for h in range(HQ_LOC):
                s = lax.dot_general(
                    q[:, h * DHP:(h + 1) * DHP], kt_ref[b, h, :, :],
                    (((1,), (0,)), ((), ())),
                    preferred_element_type=jnp.float32)
                w = jnp.exp(s + bias).astype(f16)
                cd = jnp.dot(w, va_ref[b, h, :, :],
                             preferred_element_type=jnp.float32)
                ctx_ref[:, h * DHP:(h + 1) * DHP] = (
                    cd / cd[:, DH:DH + 1]).astype(f16)
            acc_ref[pl.ds(b * SQ, SQ), :] = jnp.dot(
                ctx_ref[:, :], wo_ref[:, :],
                preferred_element_type=jnp.float32)

            for d in range(1, N_DEV):
                c = lax.rem(my + d, N_DEV)
                if b == 0:
                    @pl.when(c < 2)
                    def _(d=d):
                        rs_stage_and_send(d)
                else:
                    @pl.when(c >= 2)
                    def _(d=d):
                        rs_stage_and_send(d)

        for d in range(1, N_DEV):
            rs_send(d).wait_recv()
        red = acc_ref[pl.ds(my * CHUNK, CHUNK), :]
        for d in range(1, N_DEV):
            red = red + rs_buf[d - 1, :, :].astype(jnp.float32)
        ag_ref[pl.ds(my * CHUNK, CHUNK), :] = red.astype(f16)

        ag_rdmas = []
        for d in range(1, N_DEV):
            r = pltpu.make_async_remote_copy(
                src_ref=ag_ref.at[pl.ds(my * CHUNK, CHUNK), :],
                dst_ref=ag_ref.at[pl.ds(my * CHUNK, CHUNK), :],
                send_sem=send_sems.at[N_DEV - 1 + d - 1],
                recv_sem=recv_sems.at[N_DEV - 1 + d - 1],
                device_id=(lax.rem(my + d, N_DEV),),
                device_id_type=pl.DeviceIdType.MESH,
            )
            r.start()
            ag_rdmas.append(r)

        for d in range(1, N_DEV):
            src = lax.rem(my - d + N_DEV, N_DEV)
            pltpu.make_async_remote_copy(
                src_ref=ag_ref.at[pl.ds(src * CHUNK, CHUNK), :],
                dst_ref=ag_ref.at[pl.ds(src * CHUNK, CHUNK), :],
                send_sem=send_sems.at[N_DEV - 1 + d - 1],
                recv_sem=recv_sems.at[N_DEV - 1 + d - 1],
                device_id=(src,),
                device_id_type=pl.DeviceIdType.MESH,
            ).wait_recv()

        for d in range(1, N_DEV):
            rs_send(d).wait_send()
        for r in ag_rdmas:
            r.wait_send()

        out_ref[0, :, :] = ag_ref[pl.ds(0, SQ), :].astype(jnp.float32)
        out_ref[1, :, :] = ag_ref[pl.ds(SQ, SQ), :].astype(jnp.float32)

    return pl.pallas_call(
        body,
        out_shape=jax.ShapeDtypeStruct((B, SQ, DM), jnp.float32),
        in_specs=[pl.BlockSpec(memory_space=pltpu.VMEM)] * 5,
        out_specs=pl.BlockSpec(memory_space=pltpu.VMEM),
        scratch_shapes=[
            pltpu.VMEM((ROWS, DM), jnp.float32),
            pltpu.VMEM((SQ, DQP), jnp.bfloat16),
            pltpu.VMEM((N_DEV - 1, CHUNK, DM), jnp.bfloat16),
            pltpu.VMEM((N_DEV - 1, CHUNK, DM), jnp.bfloat16),
            pltpu.VMEM((ROWS, DM), jnp.bfloat16),
            pltpu.SemaphoreType.DMA((2 * (N_DEV - 1),)),
            pltpu.SemaphoreType.DMA((2 * (N_DEV - 1),)),
        ],
        compiler_params=pltpu.CompilerParams(collective_id=0),
    )(x16, Wq_pad, Kt_pad, Vaug, Wo_pad)
